# Initial kernel scaffold; baseline (speedup 1.0000x reference)
import jax
import jax.numpy as jnp
from jax.experimental import pallas as pl


def kernel(x, edge_index, W1, b1, W2, b2, W3, b3):
    raise NotImplementedError("write your pallas kernel here")


# speedup vs baseline: 10.1406x; 10.1406x over previous
import functools

import jax
import jax.numpy as jnp
from jax import lax
from jax.experimental import pallas as pl
from jax.experimental.pallas import tpu as pltpu
from jax.experimental.pallas import tpu_sc as plsc

_N = 10000
_E = 320000
_EB = 80
_NSUB = 16
_NCORE = 2


def _zero_fill_vmem(ref, nwords):
    zero = jnp.zeros((16,), jnp.float32)

    def body(i, carry):
        ref[pl.ds(i * 16, 16)] = zero
        return carry

    lax.fori_loop(0, nwords // 16, body, 0)


_CHUNK = 80


def _tile_chunks(s):
    base = s * 640
    nchunks = jnp.where(s < 15, 8, 5)
    return base, nchunks


def _deg_kernel_body(dst_hbm, deg0_hbm, deg1_hbm, didx, ones, zbuf, acc):
    c = lax.axis_index("c")
    s = lax.axis_index("s")
    one = jnp.full((16,), 1.0, jnp.float32)
    for j in range(_EB // 16):
        ones[pl.ds(j * 16, 16)] = one
    _zero_fill_vmem(zbuf, _CHUNK)
    row0, nchunks = _tile_chunks(s)

    def zcopy(k, carry):
        pltpu.sync_copy(zbuf, acc.at[pl.ds(row0 + k * _CHUNK, _CHUNK)])
        return carry

    lax.fori_loop(0, nchunks, zcopy, 0)
    plsc.subcore_barrier()
    eps = _E // (_NCORE * _NSUB)
    base = c * (_E // _NCORE) + s * eps

    def body(i, carry):
        pltpu.sync_copy(dst_hbm.at[pl.ds(base + i * _EB, _EB)], didx)
        pltpu.sync_copy(ones, acc.at[didx], add=True)
        return carry

    lax.fori_loop(0, eps // _EB, body, 0)
    plsc.subcore_barrier()

    @pl.when(c == 0)
    def _():
        def wb(k, carry):
            sl = pl.ds(row0 + k * _CHUNK, _CHUNK)
            pltpu.sync_copy(acc.at[sl], zbuf)
            pltpu.sync_copy(zbuf, deg0_hbm.at[sl])
            return carry
        lax.fori_loop(0, nchunks, wb, 0)

    @pl.when(c == 1)
    def _():
        def wb(k, carry):
            sl = pl.ds(row0 + k * _CHUNK, _CHUNK)
            pltpu.sync_copy(acc.at[sl], zbuf)
            pltpu.sync_copy(zbuf, deg1_hbm.at[sl])
            return carry
        lax.fori_loop(0, nchunks, wb, 0)


@functools.partial(
    pl.kernel,
    out_type=(jax.ShapeDtypeStruct((_N,), jnp.float32),
              jax.ShapeDtypeStruct((_N,), jnp.float32)),
    mesh=plsc.VectorSubcoreMesh(core_axis_name="c", subcore_axis_name="s"),
    scratch_types=(
        pltpu.VMEM((_EB,), jnp.int32),
        pltpu.VMEM((_EB,), jnp.float32),
        pltpu.VMEM((_CHUNK,), jnp.float32),
        pltpu.VMEM_SHARED((_N,), jnp.float32),
    ),
)
def _deg_kernel(dst_hbm, deg0_hbm, deg1_hbm, didx, ones, zbuf, acc):
    _deg_kernel_body(dst_hbm, deg0_hbm, deg1_hbm, didx, ones, zbuf, acc)


_DC = 128


def _make_seg_sum(mode):
    dc = _DC
    eps = _E // _NSUB if mode == "col" else _E // (_NCORE * _NSUB)

    def body(h0, h1, src_hbm, dst_hbm, out0, out1, sidx, didx, rows, zbuf,
             acc, sem):
        c = lax.axis_index("c")
        s = lax.axis_index("s")
        zero = jnp.zeros((16,), jnp.float32)

        def zfill(r, carry):
            for j in range(dc // 16):
                zbuf[r, pl.ds(j * 16, 16)] = zero
            return carry

        lax.fori_loop(0, _CHUNK, zfill, 0)
        row0, nchunks = _tile_chunks(s)

        def zcopy(k, carry):
            pltpu.sync_copy(zbuf, acc.at[pl.ds(row0 + k * _CHUNK, _CHUNK)])
            return carry

        lax.fori_loop(0, nchunks, zcopy, 0)
        plsc.subcore_barrier()

        if mode == "col":
            base = s * eps
        else:
            base = c * (_E // _NCORE) + s * eps

        def step(i, carry):
            off = base + i * _EB
            pltpu.sync_copy(src_hbm.at[pl.ds(off, _EB)], sidx)
            pltpu.sync_copy(dst_hbm.at[pl.ds(off, _EB)], didx)

            if mode == "col":
                @pl.when(c == 0)
                def _():
                    pltpu.async_copy(h0.at[sidx], rows, sem).wait()

                @pl.when(c == 1)
                def _():
                    pltpu.async_copy(h1.at[sidx], rows, sem).wait()
            else:
                pltpu.async_copy(h0.at[sidx], rows, sem).wait()

            pltpu.sync_copy(rows, acc.at[didx], add=True)
            return carry

        lax.fori_loop(0, eps // _EB, step, 0)
        plsc.subcore_barrier()

        @pl.when(c == 0)
        def _():
            def wb(k, carry):
                sl = pl.ds(row0 + k * _CHUNK, _CHUNK)
                pltpu.sync_copy(acc.at[sl], zbuf)
                pltpu.sync_copy(zbuf, out0.at[sl])
                return carry
            lax.fori_loop(0, nchunks, wb, 0)

        @pl.when(c == 1)
        def _():
            def wb(k, carry):
                sl = pl.ds(row0 + k * _CHUNK, _CHUNK)
                pltpu.sync_copy(acc.at[sl], zbuf)
                pltpu.sync_copy(zbuf, out1.at[sl])
                return carry
            lax.fori_loop(0, nchunks, wb, 0)

    return pl.kernel(
        body,
        out_type=(jax.ShapeDtypeStruct((_N, dc), jnp.float32),
                  jax.ShapeDtypeStruct((_N, dc), jnp.float32)),
        mesh=plsc.VectorSubcoreMesh(core_axis_name="c", subcore_axis_name="s"),
        scratch_types=(
            pltpu.VMEM((_EB,), jnp.int32),
            pltpu.VMEM((_EB,), jnp.int32),
            pltpu.VMEM((_EB, dc), jnp.float32),
            pltpu.VMEM((_CHUNK, dc), jnp.float32),
            pltpu.VMEM_SHARED((_N, dc), jnp.float32),
            pltpu.SemaphoreType.DMA,
        ),
    )


_seg_sum_col = _make_seg_sum("col")
_seg_sum_edge = _make_seg_sum("edge")


_BN = 1000


def _scale_body(deg0, deg1, x, dinv, xs):
    deg = deg0[...] + deg1[...] + 1.0
    di = lax.rsqrt(deg)
    dinv[...] = di
    xs[...] = di * x[...]


def _scale(deg0, deg1, x):
    return pl.pallas_call(
        _scale_body,
        grid=(_N // _BN,),
        in_specs=[
            pl.BlockSpec((_BN, 1), lambda i: (i, 0)),
            pl.BlockSpec((_BN, 1), lambda i: (i, 0)),
            pl.BlockSpec((_BN, 128), lambda i: (i, 0)),
        ],
        out_specs=[
            pl.BlockSpec((_BN, 1), lambda i: (i, 0)),
            pl.BlockSpec((_BN, 128), lambda i: (i, 0)),
        ],
        out_shape=[
            jax.ShapeDtypeStruct((_N, 1), jnp.float32),
            jax.ShapeDtypeStruct((_N, 128), jnp.float32),
        ],
    )(deg0, deg1, x)


def _zmm_body(agg0, agg1, selfh, dinv, w, b, z, colsum, sumsq, pre_matmul,
              combine):
    i = pl.program_id(0)
    di = dinv[...]
    if combine == "concat":
        agg = jnp.concatenate([agg0[...], agg1[...]], axis=1)
    else:
        agg = agg0[...] + agg1[...]
    pre = di * agg + di * selfh[...]
    if pre_matmul:
        zi = jnp.dot(pre, w[...], preferred_element_type=jnp.float32) + b[...]
    else:
        zi = pre + b[...]
    z[...] = zi

    @pl.when(i == 0)
    def _():
        colsum[...] = jnp.zeros_like(colsum)
        sumsq[...] = jnp.zeros_like(sumsq)

    colsum[...] += jnp.sum(zi, axis=0, keepdims=True)
    sumsq[...] += jnp.sum(zi * zi, keepdims=True).reshape(1, 1)


def _zmm(agg0, agg1, selfh, dinv, w, b, d_out, pre_matmul, combine):
    dc = agg0.shape[1]
    din = selfh.shape[1]
    body = functools.partial(_zmm_body, pre_matmul=pre_matmul,
                             combine=combine)
    return pl.pallas_call(
        body,
        grid=(_N // _BN,),
        in_specs=[
            pl.BlockSpec((_BN, dc), lambda i: (i, 0)),
            pl.BlockSpec((_BN, dc), lambda i: (i, 0)),
            pl.BlockSpec((_BN, din), lambda i: (i, 0)),
            pl.BlockSpec((_BN, 1), lambda i: (i, 0)),
            pl.BlockSpec(w.shape, lambda i: (0, 0)),
            pl.BlockSpec((1, d_out), lambda i: (0, 0)),
        ],
        out_specs=[
            pl.BlockSpec((_BN, d_out), lambda i: (i, 0)),
            pl.BlockSpec((1, d_out), lambda i: (0, 0)),
            pl.BlockSpec((1, 1), lambda i: (0, 0)),
        ],
        out_shape=[
            jax.ShapeDtypeStruct((_N, d_out), jnp.float32),
            jax.ShapeDtypeStruct((1, d_out), jnp.float32),
            jax.ShapeDtypeStruct((1, 1), jnp.float32),
        ],
    )(agg0, agg1, selfh, dinv, w, b)


def _norm_mm_body(z, colsum, sumsq, dinv, w, *outs, split):
    mu = colsum[...] / _N
    var = sumsq[0, 0] / _N - jnp.sum(mu * mu)
    s = lax.rsqrt(1e-6 + var)
    h = jnp.maximum((z[...] - mu) * s, 0.0)
    t = jnp.dot(h, w[...], preferred_element_type=jnp.float32)
    hs = dinv[...] * t
    if split:
        half = t.shape[1] // 2
        outs[0][...] = hs[:, :half]
        outs[1][...] = hs[:, half:]
    else:
        outs[0][...] = hs


def _norm_mm(z, colsum, sumsq, dinv, w, split):
    d_in = z.shape[1]
    d_out = w.shape[1]
    if split:
        out_specs = [pl.BlockSpec((_BN, d_out // 2), lambda i: (i, 0)),
                     pl.BlockSpec((_BN, d_out // 2), lambda i: (i, 0))]
        out_shape = [jax.ShapeDtypeStruct((_N, d_out // 2), jnp.float32),
                     jax.ShapeDtypeStruct((_N, d_out // 2), jnp.float32)]
    else:
        out_specs = [pl.BlockSpec((_BN, d_out), lambda i: (i, 0))]
        out_shape = [jax.ShapeDtypeStruct((_N, d_out), jnp.float32)]
    return pl.pallas_call(
        functools.partial(_norm_mm_body, split=split),
        grid=(_N // _BN,),
        in_specs=[
            pl.BlockSpec((_BN, d_in), lambda i: (i, 0)),
            pl.BlockSpec((1, d_in), lambda i: (0, 0)),
            pl.BlockSpec((1, 1), lambda i: (0, 0)),
            pl.BlockSpec((_BN, 1), lambda i: (i, 0)),
            pl.BlockSpec(w.shape, lambda i: (0, 0)),
        ],
        out_specs=out_specs,
        out_shape=out_shape,
    )(z, colsum, sumsq, dinv, w)


def _final_body(p0, p1, hs, dinv, b, out):
    di = dinv[...]
    out[...] = di * (p0[...] + p1[...]) + di * hs[...] + b[...]


def _final(p0, p1, hs, dinv, b):
    dc = p0.shape[1]
    return pl.pallas_call(
        _final_body,
        grid=(_N // _BN,),
        in_specs=[
            pl.BlockSpec((_BN, dc), lambda i: (i, 0)),
            pl.BlockSpec((_BN, dc), lambda i: (i, 0)),
            pl.BlockSpec((_BN, dc), lambda i: (i, 0)),
            pl.BlockSpec((_BN, 1), lambda i: (i, 0)),
            pl.BlockSpec((1, dc), lambda i: (0, 0)),
        ],
        out_specs=pl.BlockSpec((_BN, dc), lambda i: (i, 0)),
        out_shape=jax.ShapeDtypeStruct((_N, dc), jnp.float32),
    )(p0, p1, hs, dinv, b)


def kernel(x, edge_index, W1, b1, W2, b2, W3, b3):
    src = edge_index[0]
    dst = edge_index[1]

    deg0, deg1 = _deg_kernel(dst)
    dinv, xs = _scale(deg0.reshape(_N, 1), deg1.reshape(_N, 1), x)

    p0, p1 = _seg_sum_edge(xs, xs, src, dst)
    z1, cs1, ss1 = _zmm(p0, p1, xs, dinv, W1, b1.reshape(1, -1),
                        256, pre_matmul=True, combine="add")
    hs2_0, hs2_1 = _norm_mm(z1, cs1, ss1, dinv, W2, split=True)

    agg2_0, agg2_1 = _seg_sum_col(hs2_0, hs2_1, src, dst)
    selfh2 = jnp.concatenate([hs2_0, hs2_1], axis=1)
    z2, cs2, ss2 = _zmm(agg2_0, agg2_1, selfh2, dinv, W2,
                        b2.reshape(1, -1), 256, pre_matmul=False,
                        combine="concat")
    W3p = jnp.pad(W3, ((0, 0), (0, 88)))
    [hs3] = _norm_mm(z2, cs2, ss2, dinv, W3p, split=False)

    q0, q1 = _seg_sum_edge(hs3, hs3, src, dst)
    b3p = jnp.pad(b3, (0, 88)).reshape(1, -1)
    outp = _final(q0, q1, hs3, dinv, b3p)
    return outp[:, :40]


# speedup vs baseline: 17.6053x; 1.7361x over previous
import functools

import jax
import jax.numpy as jnp
from jax import lax
from jax.experimental import pallas as pl
from jax.experimental.pallas import tpu as pltpu
from jax.experimental.pallas import tpu_sc as plsc

_N = 10000
_E = 320000
_EB = 80
_NSUB = 16
_NCORE = 2


def _zero_fill_vmem(ref, nwords):
    zero = jnp.zeros((16,), jnp.float32)

    def body(i, carry):
        ref[pl.ds(i * 16, 16)] = zero
        return carry

    lax.fori_loop(0, nwords // 16, body, 0)


_CHUNK = 80


def _tile_chunks(s):
    base = s * 640
    nchunks = jnp.where(s < 15, 8, 5)
    return base, nchunks


def _deg_kernel_body(dst_hbm, deg0_hbm, deg1_hbm, didx, ones, zbuf, acc):
    c = lax.axis_index("c")
    s = lax.axis_index("s")
    one = jnp.full((16,), 1.0, jnp.float32)
    for j in range(_EB // 16):
        ones[pl.ds(j * 16, 16)] = one
    _zero_fill_vmem(zbuf, _CHUNK)
    row0, nchunks = _tile_chunks(s)

    def zcopy(k, carry):
        pltpu.sync_copy(zbuf, acc.at[pl.ds(row0 + k * _CHUNK, _CHUNK)])
        return carry

    lax.fori_loop(0, nchunks, zcopy, 0)
    plsc.subcore_barrier()
    eps = _E // (_NCORE * _NSUB)
    base = c * (_E // _NCORE) + s * eps

    def body(i, carry):
        pltpu.sync_copy(dst_hbm.at[pl.ds(base + i * _EB, _EB)], didx)
        pltpu.sync_copy(ones, acc.at[didx], add=True)
        return carry

    lax.fori_loop(0, eps // _EB, body, 0)
    plsc.subcore_barrier()

    @pl.when(c == 0)
    def _():
        def wb(k, carry):
            sl = pl.ds(row0 + k * _CHUNK, _CHUNK)
            pltpu.sync_copy(acc.at[sl], zbuf)
            pltpu.sync_copy(zbuf, deg0_hbm.at[sl])
            return carry
        lax.fori_loop(0, nchunks, wb, 0)

    @pl.when(c == 1)
    def _():
        def wb(k, carry):
            sl = pl.ds(row0 + k * _CHUNK, _CHUNK)
            pltpu.sync_copy(acc.at[sl], zbuf)
            pltpu.sync_copy(zbuf, deg1_hbm.at[sl])
            return carry
        lax.fori_loop(0, nchunks, wb, 0)


@functools.partial(
    pl.kernel,
    out_type=(jax.ShapeDtypeStruct((_N,), jnp.float32),
              jax.ShapeDtypeStruct((_N,), jnp.float32)),
    mesh=plsc.VectorSubcoreMesh(core_axis_name="c", subcore_axis_name="s"),
    scratch_types=(
        pltpu.VMEM((_EB,), jnp.int32),
        pltpu.VMEM((_EB,), jnp.float32),
        pltpu.VMEM((_CHUNK,), jnp.float32),
        pltpu.VMEM_SHARED((_N,), jnp.float32),
    ),
)
def _deg_kernel(dst_hbm, deg0_hbm, deg1_hbm, didx, ones, zbuf, acc):
    _deg_kernel_body(dst_hbm, deg0_hbm, deg1_hbm, didx, ones, zbuf, acc)


_DC = 128


def _make_seg_sum():
    dc = _DC
    k = 3
    nb = _E // (_NCORE * _NSUB) // _EB
    nfull = nb // k
    tail = nb - nfull * k

    def body(h0, src_hbm, dst_hbm, out0, out1, sidx, didx, rows, acc,
             *sems):
        isem = sems[:k]
        jsem = sems[k:]
        c = lax.axis_index("c")
        s = lax.axis_index("s")
        zero = jnp.zeros((16,), jnp.float32)

        def zfill(r, carry):
            for j in range(dc // 16):
                rows[0, r, pl.ds(j * 16, 16)] = zero
            return carry

        lax.fori_loop(0, _CHUNK, zfill, 0)
        row0, nchunks = _tile_chunks(s)

        def zcopy(q, carry):
            pltpu.sync_copy(rows.at[0],
                            acc.at[pl.ds(row0 + q * _CHUNK, _CHUNK)])
            return carry

        lax.fori_loop(0, nchunks, zcopy, 0)
        plsc.subcore_barrier()

        ebase = (c * (nb * _NSUB) + s * nb) * _EB

        def load_src(i, b):
            pltpu.async_copy(src_hbm.at[pl.ds(ebase + i * _EB, _EB)],
                             sidx.at[b, 0], isem[b])

        def load_dst(i, b):
            pltpu.async_copy(dst_hbm.at[pl.ds(ebase + i * _EB, _EB)],
                             didx.at[b, 0], jsem[b])

        def drain_src(b):
            pltpu.make_async_copy(src_hbm.at[pl.ds(0, _EB)], sidx.at[b, 0],
                                  isem[b]).wait()

        def drain_dst(b):
            pltpu.make_async_copy(dst_hbm.at[pl.ds(0, _EB)], didx.at[b, 0],
                                  jsem[b]).wait()

        def do_block(m, nbatch, prefetch_next):
            descs = []
            for b in range(nbatch):
                drain_src(b)
                descs.append(
                    pltpu.async_copy(h0.at[sidx.at[b, 0]], rows.at[b],
                                     isem[b]))
            for b in range(nbatch):
                descs[b].wait()
                if prefetch_next:
                    @pl.when(m * k + k + b < nb)
                    def _():
                        load_src(m * k + k + b, b)
            for b in range(nbatch):
                drain_dst(b)
                pltpu.sync_copy(rows.at[b], acc.at[didx.at[b, 0]],
                                add=True)
                if prefetch_next:
                    @pl.when(m * k + k + b < nb)
                    def _():
                        load_dst(m * k + k + b, b)

        for b in range(k):
            load_src(b, b)
            load_dst(b, b)

        def step(m, carry):
            do_block(m, k, True)
            return carry

        lax.fori_loop(0, nfull, step, 0)
        if tail:
            do_block(nfull, tail, False)
        plsc.subcore_barrier()

        @pl.when(c == 0)
        def _():
            def wb(q, carry):
                sl = pl.ds(row0 + q * _CHUNK, _CHUNK)
                pltpu.sync_copy(acc.at[sl], rows.at[0])
                pltpu.sync_copy(rows.at[0], out0.at[sl])
                return carry
            lax.fori_loop(0, nchunks, wb, 0)

        @pl.when(c == 1)
        def _():
            def wb(q, carry):
                sl = pl.ds(row0 + q * _CHUNK, _CHUNK)
                pltpu.sync_copy(acc.at[sl], rows.at[0])
                pltpu.sync_copy(rows.at[0], out1.at[sl])
                return carry
            lax.fori_loop(0, nchunks, wb, 0)

    return pl.kernel(
        body,
        out_type=(jax.ShapeDtypeStruct((_N, dc), jnp.float32),
                  jax.ShapeDtypeStruct((_N, dc), jnp.float32)),
        mesh=plsc.VectorSubcoreMesh(core_axis_name="c", subcore_axis_name="s"),
        scratch_types=(
            pltpu.VMEM((k, 1, _EB), jnp.int32),
            pltpu.VMEM((k, 1, _EB), jnp.int32),
            pltpu.VMEM((k, _EB, dc), jnp.float32),
            pltpu.VMEM_SHARED((_N, dc), jnp.float32),
        ) + (pltpu.SemaphoreType.DMA,) * (2 * k),
    )


_seg_sum = _make_seg_sum()


_BN = 1000


def _scale_body(deg0, deg1, x, dinv, xs):
    deg = deg0[...] + deg1[...] + 1.0
    di = lax.rsqrt(deg)
    dinv[...] = di
    xs[...] = di * x[...]


def _scale(deg0, deg1, x):
    return pl.pallas_call(
        _scale_body,
        grid=(_N // _BN,),
        in_specs=[
            pl.BlockSpec((_BN, 1), lambda i: (i, 0)),
            pl.BlockSpec((_BN, 1), lambda i: (i, 0)),
            pl.BlockSpec((_BN, 128), lambda i: (i, 0)),
        ],
        out_specs=[
            pl.BlockSpec((_BN, 1), lambda i: (i, 0)),
            pl.BlockSpec((_BN, 128), lambda i: (i, 0)),
        ],
        out_shape=[
            jax.ShapeDtypeStruct((_N, 1), jnp.float32),
            jax.ShapeDtypeStruct((_N, 128), jnp.float32),
        ],
    )(deg0, deg1, x)


def _zmm_body(*refs, pre_matmul, nagg):
    aggs = refs[:nagg]
    selfh, dinv, w, b, z, colsum, sumsq = refs[nagg:]
    i = pl.program_id(0)
    di = dinv[...]
    if nagg == 2:
        agg = aggs[0][...] + aggs[1][...]
    else:
        agg = jnp.concatenate([aggs[0][...] + aggs[1][...],
                               aggs[2][...] + aggs[3][...]], axis=1)
    pre = di * agg + di * selfh[...]
    if pre_matmul:
        zi = jnp.dot(pre, w[...], preferred_element_type=jnp.float32) + b[...]
    else:
        zi = pre + b[...]
    z[...] = zi

    @pl.when(i == 0)
    def _():
        colsum[...] = jnp.zeros_like(colsum)
        sumsq[...] = jnp.zeros_like(sumsq)

    colsum[...] += jnp.sum(zi, axis=0, keepdims=True)
    sumsq[...] += jnp.sum(zi * zi, keepdims=True).reshape(1, 1)


def _zmm(aggs, selfh, dinv, w, b, d_out, pre_matmul):
    dc = aggs[0].shape[1]
    din = selfh.shape[1]
    body = functools.partial(_zmm_body, pre_matmul=pre_matmul,
                             nagg=len(aggs))
    return pl.pallas_call(
        body,
        grid=(_N // _BN,),
        in_specs=[pl.BlockSpec((_BN, dc), lambda i: (i, 0))
                  for _ in aggs] + [
            pl.BlockSpec((_BN, din), lambda i: (i, 0)),
            pl.BlockSpec((_BN, 1), lambda i: (i, 0)),
            pl.BlockSpec(w.shape, lambda i: (0, 0)),
            pl.BlockSpec((1, d_out), lambda i: (0, 0)),
        ],
        out_specs=[
            pl.BlockSpec((_BN, d_out), lambda i: (i, 0)),
            pl.BlockSpec((1, d_out), lambda i: (0, 0)),
            pl.BlockSpec((1, 1), lambda i: (0, 0)),
        ],
        out_shape=[
            jax.ShapeDtypeStruct((_N, d_out), jnp.float32),
            jax.ShapeDtypeStruct((1, d_out), jnp.float32),
            jax.ShapeDtypeStruct((1, 1), jnp.float32),
        ],
    )(*aggs, selfh, dinv, w, b)


def _norm_mm_body(z, colsum, sumsq, dinv, w, *outs, split):
    mu = colsum[...] / _N
    var = sumsq[0, 0] / _N - jnp.sum(mu * mu)
    s = lax.rsqrt(1e-6 + var)
    h = jnp.maximum((z[...] - mu) * s, 0.0)
    t = jnp.dot(h, w[...], preferred_element_type=jnp.float32)
    hs = dinv[...] * t
    if split:
        half = t.shape[1] // 2
        outs[0][...] = hs[:, :half]
        outs[1][...] = hs[:, half:]
    else:
        outs[0][...] = hs


def _norm_mm(z, colsum, sumsq, dinv, w, split):
    d_in = z.shape[1]
    d_out = w.shape[1]
    if split:
        out_specs = [pl.BlockSpec((_BN, d_out // 2), lambda i: (i, 0)),
                     pl.BlockSpec((_BN, d_out // 2), lambda i: (i, 0))]
        out_shape = [jax.ShapeDtypeStruct((_N, d_out // 2), jnp.float32),
                     jax.ShapeDtypeStruct((_N, d_out // 2), jnp.float32)]
    else:
        out_specs = [pl.BlockSpec((_BN, d_out), lambda i: (i, 0))]
        out_shape = [jax.ShapeDtypeStruct((_N, d_out), jnp.float32)]
    return pl.pallas_call(
        functools.partial(_norm_mm_body, split=split),
        grid=(_N // _BN,),
        in_specs=[
            pl.BlockSpec((_BN, d_in), lambda i: (i, 0)),
            pl.BlockSpec((1, d_in), lambda i: (0, 0)),
            pl.BlockSpec((1, 1), lambda i: (0, 0)),
            pl.BlockSpec((_BN, 1), lambda i: (i, 0)),
            pl.BlockSpec(w.shape, lambda i: (0, 0)),
        ],
        out_specs=out_specs,
        out_shape=out_shape,
    )(z, colsum, sumsq, dinv, w)


def _final_body(p0, p1, hs, dinv, b, out):
    di = dinv[...]
    out[...] = di * (p0[...] + p1[...]) + di * hs[...] + b[...]


def _final(p0, p1, hs, dinv, b):
    dc = p0.shape[1]
    return pl.pallas_call(
        _final_body,
        grid=(_N // _BN,),
        in_specs=[
            pl.BlockSpec((_BN, dc), lambda i: (i, 0)),
            pl.BlockSpec((_BN, dc), lambda i: (i, 0)),
            pl.BlockSpec((_BN, dc), lambda i: (i, 0)),
            pl.BlockSpec((_BN, 1), lambda i: (i, 0)),
            pl.BlockSpec((1, dc), lambda i: (0, 0)),
        ],
        out_specs=pl.BlockSpec((_BN, dc), lambda i: (i, 0)),
        out_shape=jax.ShapeDtypeStruct((_N, dc), jnp.float32),
    )(p0, p1, hs, dinv, b)


def kernel(x, edge_index, W1, b1, W2, b2, W3, b3):
    src = edge_index[0]
    dst = edge_index[1]
    src2 = src.reshape(_E // _EB, 1, _EB)
    dst2 = dst.reshape(_E // _EB, 1, _EB)

    deg0, deg1 = _deg_kernel(dst)
    dinv, xs = _scale(deg0.reshape(_N, 1), deg1.reshape(_N, 1), x)

    p0, p1 = _seg_sum(xs, src, dst)
    z1, cs1, ss1 = _zmm((p0, p1), xs, dinv, W1, b1.reshape(1, -1),
                        256, pre_matmul=True)
    hs2_0, hs2_1 = _norm_mm(z1, cs1, ss1, dinv, W2, split=True)

    a0, a1 = _seg_sum(hs2_0, src, dst)
    hs2_1d, a0, a1 = lax.optimization_barrier((hs2_1, a0, a1))
    a2, a3 = _seg_sum(hs2_1d, src, dst)
    selfh2 = jnp.concatenate([hs2_0, hs2_1], axis=1)
    z2, cs2, ss2 = _zmm((a0, a1, a2, a3), selfh2, dinv, W2,
                        b2.reshape(1, -1), 256, pre_matmul=False)
    W3p = jnp.pad(W3, ((0, 0), (0, 88)))
    [hs3] = _norm_mm(z2, cs2, ss2, dinv, W3p, split=False)

    q0, q1 = _seg_sum(hs3, src, dst)
    b3p = jnp.pad(b3, (0, 88)).reshape(1, -1)
    outp = _final(q0, q1, hs3, dinv, b3p)
    return outp[:, :40]


# speedup vs baseline: 18.0139x; 1.0232x over previous
import functools

import jax
import jax.numpy as jnp
from jax import lax
from jax.experimental import pallas as pl
from jax.experimental.pallas import tpu as pltpu
from jax.experimental.pallas import tpu_sc as plsc

_N = 10000
_E = 320000
_EB = 80
_NSUB = 16
_NCORE = 2


def _zero_fill_vmem(ref, nwords):
    zero = jnp.zeros((16,), jnp.float32)

    def body(i, carry):
        ref[pl.ds(i * 16, 16)] = zero
        return carry

    lax.fori_loop(0, nwords // 16, body, 0)


_CHUNK = 80


def _tile_chunks(s):
    base = s * 640
    nchunks = jnp.where(s < 15, 8, 5)
    return base, nchunks


def _deg_kernel_body(dst_hbm, deg0_hbm, deg1_hbm, didx, ones, zbuf, acc):
    c = lax.axis_index("c")
    s = lax.axis_index("s")
    one = jnp.full((16,), 1.0, jnp.float32)
    for j in range(_EB // 16):
        ones[pl.ds(j * 16, 16)] = one
    _zero_fill_vmem(zbuf, _CHUNK)
    row0, nchunks = _tile_chunks(s)

    def zcopy(k, carry):
        pltpu.sync_copy(zbuf, acc.at[pl.ds(row0 + k * _CHUNK, _CHUNK)])
        return carry

    lax.fori_loop(0, nchunks, zcopy, 0)
    plsc.subcore_barrier()
    eps = _E // (_NCORE * _NSUB)
    base = c * (_E // _NCORE) + s * eps

    def body(i, carry):
        pltpu.sync_copy(dst_hbm.at[pl.ds(base + i * _EB, _EB)], didx)
        pltpu.sync_copy(ones, acc.at[didx], add=True)
        return carry

    lax.fori_loop(0, eps // _EB, body, 0)
    plsc.subcore_barrier()

    @pl.when(c == 0)
    def _():
        def wb(k, carry):
            sl = pl.ds(row0 + k * _CHUNK, _CHUNK)
            pltpu.sync_copy(acc.at[sl], zbuf)
            pltpu.sync_copy(zbuf, deg0_hbm.at[sl])
            return carry
        lax.fori_loop(0, nchunks, wb, 0)

    @pl.when(c == 1)
    def _():
        def wb(k, carry):
            sl = pl.ds(row0 + k * _CHUNK, _CHUNK)
            pltpu.sync_copy(acc.at[sl], zbuf)
            pltpu.sync_copy(zbuf, deg1_hbm.at[sl])
            return carry
        lax.fori_loop(0, nchunks, wb, 0)


@functools.partial(
    pl.kernel,
    out_type=(jax.ShapeDtypeStruct((_N,), jnp.float32),
              jax.ShapeDtypeStruct((_N,), jnp.float32)),
    mesh=plsc.VectorSubcoreMesh(core_axis_name="c", subcore_axis_name="s"),
    scratch_types=(
        pltpu.VMEM((_EB,), jnp.int32),
        pltpu.VMEM((_EB,), jnp.float32),
        pltpu.VMEM((_CHUNK,), jnp.float32),
        pltpu.VMEM_SHARED((_N,), jnp.float32),
    ),
)
def _deg_kernel(dst_hbm, deg0_hbm, deg1_hbm, didx, ones, zbuf, acc):
    _deg_kernel_body(dst_hbm, deg0_hbm, deg1_hbm, didx, ones, zbuf, acc)


_DC = 128


def _make_seg_sum():
    dc = _DC
    k = 3
    nb = _E // (_NCORE * _NSUB) // _EB
    nfull = nb // k
    tail = nb - nfull * k

    def body(h0, src_hbm, dst_hbm, out0, out1, sidx, didx, rows, acc,
             *sems):
        isem = sems[:k]
        jsem = sems[k:]
        c = lax.axis_index("c")
        s = lax.axis_index("s")
        zero = jnp.zeros((16,), jnp.float32)

        def zfill(r, carry):
            for j in range(dc // 16):
                rows[0, r, pl.ds(j * 16, 16)] = zero
            return carry

        lax.fori_loop(0, _CHUNK, zfill, 0)
        row0, nchunks = _tile_chunks(s)

        def zcopy(q, carry):
            pltpu.sync_copy(rows.at[0],
                            acc.at[pl.ds(row0 + q * _CHUNK, _CHUNK)])
            return carry

        lax.fori_loop(0, nchunks, zcopy, 0)
        plsc.subcore_barrier()

        ebase = (c * (nb * _NSUB) + s * nb) * _EB

        def load_src(i, b):
            pltpu.async_copy(src_hbm.at[pl.ds(ebase + i * _EB, _EB)],
                             sidx.at[b, 0], isem[b])

        def load_dst(i, b):
            pltpu.async_copy(dst_hbm.at[pl.ds(ebase + i * _EB, _EB)],
                             didx.at[b, 0], jsem[b])

        def drain_src(b):
            pltpu.make_async_copy(src_hbm.at[pl.ds(0, _EB)], sidx.at[b, 0],
                                  isem[b]).wait()

        def drain_dst(b):
            pltpu.make_async_copy(dst_hbm.at[pl.ds(0, _EB)], didx.at[b, 0],
                                  jsem[b]).wait()

        def do_block(m, nbatch, prefetch_next):
            descs = []
            for b in range(nbatch):
                drain_src(b)
                descs.append(
                    pltpu.async_copy(h0.at[sidx.at[b, 0]], rows.at[b],
                                     isem[b]))
            for b in range(nbatch):
                descs[b].wait()
                if prefetch_next:
                    @pl.when(m * k + k + b < nb)
                    def _():
                        load_src(m * k + k + b, b)
            sdescs = []
            for b in range(nbatch):
                drain_dst(b)
                sdescs.append(
                    pltpu.async_copy(rows.at[b], acc.at[didx.at[b, 0]],
                                     jsem[b], add=True))
            for b in range(nbatch):
                sdescs[b].wait()
                if prefetch_next:
                    @pl.when(m * k + k + b < nb)
                    def _():
                        load_dst(m * k + k + b, b)

        for b in range(k):
            load_src(b, b)
            load_dst(b, b)

        def step(m, carry):
            do_block(m, k, True)
            return carry

        lax.fori_loop(0, nfull, step, 0)
        if tail:
            do_block(nfull, tail, False)
        plsc.subcore_barrier()

        @pl.when(c == 0)
        def _():
            def wb(q, carry):
                sl = pl.ds(row0 + q * _CHUNK, _CHUNK)
                pltpu.sync_copy(acc.at[sl], rows.at[0])
                pltpu.sync_copy(rows.at[0], out0.at[sl])
                return carry
            lax.fori_loop(0, nchunks, wb, 0)

        @pl.when(c == 1)
        def _():
            def wb(q, carry):
                sl = pl.ds(row0 + q * _CHUNK, _CHUNK)
                pltpu.sync_copy(acc.at[sl], rows.at[0])
                pltpu.sync_copy(rows.at[0], out1.at[sl])
                return carry
            lax.fori_loop(0, nchunks, wb, 0)

    return pl.kernel(
        body,
        out_type=(jax.ShapeDtypeStruct((_N, dc), jnp.float32),
                  jax.ShapeDtypeStruct((_N, dc), jnp.float32)),
        mesh=plsc.VectorSubcoreMesh(core_axis_name="c", subcore_axis_name="s"),
        scratch_types=(
            pltpu.VMEM((k, 1, _EB), jnp.int32),
            pltpu.VMEM((k, 1, _EB), jnp.int32),
            pltpu.VMEM((k, _EB, dc), jnp.float32),
            pltpu.VMEM_SHARED((_N, dc), jnp.float32),
        ) + (pltpu.SemaphoreType.DMA,) * (2 * k),
    )


_seg_sum = _make_seg_sum()


_BN = 1000


def _scale_body(deg0, deg1, x, dinv, xs):
    deg = deg0[...] + deg1[...] + 1.0
    di = lax.rsqrt(deg)
    dinv[...] = di
    xs[...] = di * x[...]


def _scale(deg0, deg1, x):
    return pl.pallas_call(
        _scale_body,
        grid=(_N // _BN,),
        in_specs=[
            pl.BlockSpec((_BN, 1), lambda i: (i, 0)),
            pl.BlockSpec((_BN, 1), lambda i: (i, 0)),
            pl.BlockSpec((_BN, 128), lambda i: (i, 0)),
        ],
        out_specs=[
            pl.BlockSpec((_BN, 1), lambda i: (i, 0)),
            pl.BlockSpec((_BN, 128), lambda i: (i, 0)),
        ],
        out_shape=[
            jax.ShapeDtypeStruct((_N, 1), jnp.float32),
            jax.ShapeDtypeStruct((_N, 128), jnp.float32),
        ],
    )(deg0, deg1, x)


def _zmm_body(*refs, pre_matmul, nagg):
    aggs = refs[:nagg]
    selfh, dinv, w, b, z, colsum, sumsq = refs[nagg:]
    i = pl.program_id(0)
    di = dinv[...]
    if nagg == 2:
        agg = aggs[0][...] + aggs[1][...]
    else:
        agg = jnp.concatenate([aggs[0][...] + aggs[1][...],
                               aggs[2][...] + aggs[3][...]], axis=1)
    pre = di * agg + di * selfh[...]
    if pre_matmul:
        zi = jnp.dot(pre, w[...], preferred_element_type=jnp.float32) + b[...]
    else:
        zi = pre + b[...]
    z[...] = zi

    @pl.when(i == 0)
    def _():
        colsum[...] = jnp.zeros_like(colsum)
        sumsq[...] = jnp.zeros_like(sumsq)

    colsum[...] += jnp.sum(zi, axis=0, keepdims=True)
    sumsq[...] += jnp.sum(zi * zi, keepdims=True).reshape(1, 1)


def _zmm(aggs, selfh, dinv, w, b, d_out, pre_matmul):
    dc = aggs[0].shape[1]
    din = selfh.shape[1]
    body = functools.partial(_zmm_body, pre_matmul=pre_matmul,
                             nagg=len(aggs))
    return pl.pallas_call(
        body,
        grid=(_N // _BN,),
        in_specs=[pl.BlockSpec((_BN, dc), lambda i: (i, 0))
                  for _ in aggs] + [
            pl.BlockSpec((_BN, din), lambda i: (i, 0)),
            pl.BlockSpec((_BN, 1), lambda i: (i, 0)),
            pl.BlockSpec(w.shape, lambda i: (0, 0)),
            pl.BlockSpec((1, d_out), lambda i: (0, 0)),
        ],
        out_specs=[
            pl.BlockSpec((_BN, d_out), lambda i: (i, 0)),
            pl.BlockSpec((1, d_out), lambda i: (0, 0)),
            pl.BlockSpec((1, 1), lambda i: (0, 0)),
        ],
        out_shape=[
            jax.ShapeDtypeStruct((_N, d_out), jnp.float32),
            jax.ShapeDtypeStruct((1, d_out), jnp.float32),
            jax.ShapeDtypeStruct((1, 1), jnp.float32),
        ],
    )(*aggs, selfh, dinv, w, b)


def _norm_mm_body(z, colsum, sumsq, dinv, w, *outs, split):
    mu = colsum[...] / _N
    var = sumsq[0, 0] / _N - jnp.sum(mu * mu)
    s = lax.rsqrt(1e-6 + var)
    h = jnp.maximum((z[...] - mu) * s, 0.0)
    t = jnp.dot(h, w[...], preferred_element_type=jnp.float32)
    hs = dinv[...] * t
    if split:
        half = t.shape[1] // 2
        outs[0][...] = hs[:, :half]
        outs[1][...] = hs[:, half:]
    else:
        outs[0][...] = hs


def _norm_mm(z, colsum, sumsq, dinv, w, split):
    d_in = z.shape[1]
    d_out = w.shape[1]
    if split:
        out_specs = [pl.BlockSpec((_BN, d_out // 2), lambda i: (i, 0)),
                     pl.BlockSpec((_BN, d_out // 2), lambda i: (i, 0))]
        out_shape = [jax.ShapeDtypeStruct((_N, d_out // 2), jnp.float32),
                     jax.ShapeDtypeStruct((_N, d_out // 2), jnp.float32)]
    else:
        out_specs = [pl.BlockSpec((_BN, d_out), lambda i: (i, 0))]
        out_shape = [jax.ShapeDtypeStruct((_N, d_out), jnp.float32)]
    return pl.pallas_call(
        functools.partial(_norm_mm_body, split=split),
        grid=(_N // _BN,),
        in_specs=[
            pl.BlockSpec((_BN, d_in), lambda i: (i, 0)),
            pl.BlockSpec((1, d_in), lambda i: (0, 0)),
            pl.BlockSpec((1, 1), lambda i: (0, 0)),
            pl.BlockSpec((_BN, 1), lambda i: (i, 0)),
            pl.BlockSpec(w.shape, lambda i: (0, 0)),
        ],
        out_specs=out_specs,
        out_shape=out_shape,
    )(z, colsum, sumsq, dinv, w)


def _final_body(p0, p1, hs, dinv, b, out):
    di = dinv[...]
    out[...] = di * (p0[...] + p1[...]) + di * hs[...] + b[...]


def _final(p0, p1, hs, dinv, b):
    dc = p0.shape[1]
    return pl.pallas_call(
        _final_body,
        grid=(_N // _BN,),
        in_specs=[
            pl.BlockSpec((_BN, dc), lambda i: (i, 0)),
            pl.BlockSpec((_BN, dc), lambda i: (i, 0)),
            pl.BlockSpec((_BN, dc), lambda i: (i, 0)),
            pl.BlockSpec((_BN, 1), lambda i: (i, 0)),
            pl.BlockSpec((1, dc), lambda i: (0, 0)),
        ],
        out_specs=pl.BlockSpec((_BN, dc), lambda i: (i, 0)),
        out_shape=jax.ShapeDtypeStruct((_N, dc), jnp.float32),
    )(p0, p1, hs, dinv, b)


def kernel(x, edge_index, W1, b1, W2, b2, W3, b3):
    src = edge_index[0]
    dst = edge_index[1]
    src2 = src.reshape(_E // _EB, 1, _EB)
    dst2 = dst.reshape(_E // _EB, 1, _EB)

    deg0, deg1 = _deg_kernel(dst)
    dinv, xs = _scale(deg0.reshape(_N, 1), deg1.reshape(_N, 1), x)

    p0, p1 = _seg_sum(xs, src, dst)
    z1, cs1, ss1 = _zmm((p0, p1), xs, dinv, W1, b1.reshape(1, -1),
                        256, pre_matmul=True)
    hs2_0, hs2_1 = _norm_mm(z1, cs1, ss1, dinv, W2, split=True)

    a0, a1 = _seg_sum(hs2_0, src, dst)
    hs2_1d, a0, a1 = lax.optimization_barrier((hs2_1, a0, a1))
    a2, a3 = _seg_sum(hs2_1d, src, dst)
    selfh2 = jnp.concatenate([hs2_0, hs2_1], axis=1)
    z2, cs2, ss2 = _zmm((a0, a1, a2, a3), selfh2, dinv, W2,
                        b2.reshape(1, -1), 256, pre_matmul=False)
    W3p = jnp.pad(W3, ((0, 0), (0, 88)))
    [hs3] = _norm_mm(z2, cs2, ss2, dinv, W3p, split=False)

    q0, q1 = _seg_sum(hs3, src, dst)
    b3p = jnp.pad(b3, (0, 88)).reshape(1, -1)
    outp = _final(q0, q1, hs3, dinv, b3p)
    return outp[:, :40]


# speedup vs baseline: 18.3107x; 1.0165x over previous
import functools

import jax
import jax.numpy as jnp
from jax import lax
from jax.experimental import pallas as pl
from jax.experimental.pallas import tpu as pltpu
from jax.experimental.pallas import tpu_sc as plsc

_N = 10000
_E = 320000
_EB = 80
_NSUB = 16
_NCORE = 2


def _zero_fill_vmem(ref, nwords):
    zero = jnp.zeros((16,), jnp.float32)

    def body(i, carry):
        ref[pl.ds(i * 16, 16)] = zero
        return carry

    lax.fori_loop(0, nwords // 16, body, 0)


_CHUNK = 80


def _tile_chunks(s):
    base = s * 640
    nchunks = jnp.where(s < 15, 8, 5)
    return base, nchunks


def _deg_kernel_body(dst_hbm, deg0_hbm, deg1_hbm, didx, ones, zbuf, acc):
    c = lax.axis_index("c")
    s = lax.axis_index("s")
    one = jnp.full((16,), 1.0, jnp.float32)
    for j in range(_EB // 16):
        ones[pl.ds(j * 16, 16)] = one
    _zero_fill_vmem(zbuf, _CHUNK)
    row0, nchunks = _tile_chunks(s)

    def zcopy(k, carry):
        pltpu.sync_copy(zbuf, acc.at[pl.ds(row0 + k * _CHUNK, _CHUNK)])
        return carry

    lax.fori_loop(0, nchunks, zcopy, 0)
    plsc.subcore_barrier()
    eps = _E // (_NCORE * _NSUB)
    base = c * (_E // _NCORE) + s * eps

    def body(i, carry):
        pltpu.sync_copy(dst_hbm.at[pl.ds(base + i * _EB, _EB)], didx)
        pltpu.sync_copy(ones, acc.at[didx], add=True)
        return carry

    lax.fori_loop(0, eps // _EB, body, 0)
    plsc.subcore_barrier()

    @pl.when(c == 0)
    def _():
        def wb(k, carry):
            sl = pl.ds(row0 + k * _CHUNK, _CHUNK)
            pltpu.sync_copy(acc.at[sl], zbuf)
            pltpu.sync_copy(zbuf, deg0_hbm.at[sl])
            return carry
        lax.fori_loop(0, nchunks, wb, 0)

    @pl.when(c == 1)
    def _():
        def wb(k, carry):
            sl = pl.ds(row0 + k * _CHUNK, _CHUNK)
            pltpu.sync_copy(acc.at[sl], zbuf)
            pltpu.sync_copy(zbuf, deg1_hbm.at[sl])
            return carry
        lax.fori_loop(0, nchunks, wb, 0)


@functools.partial(
    pl.kernel,
    out_type=(jax.ShapeDtypeStruct((_N,), jnp.float32),
              jax.ShapeDtypeStruct((_N,), jnp.float32)),
    mesh=plsc.VectorSubcoreMesh(core_axis_name="c", subcore_axis_name="s"),
    scratch_types=(
        pltpu.VMEM((_EB,), jnp.int32),
        pltpu.VMEM((_EB,), jnp.float32),
        pltpu.VMEM((_CHUNK,), jnp.float32),
        pltpu.VMEM_SHARED((_N,), jnp.float32),
    ),
)
def _deg_kernel(dst_hbm, deg0_hbm, deg1_hbm, didx, ones, zbuf, acc):
    _deg_kernel_body(dst_hbm, deg0_hbm, deg1_hbm, didx, ones, zbuf, acc)


_DC = 128


def _make_seg_sum():
    dc = _DC
    k = 4
    nb = _E // (_NCORE * _NSUB) // _EB
    nfull = nb // k
    tail = nb - nfull * k

    def body(h0, src_hbm, dst_hbm, out0, out1, sidx, didx, rows, acc,
             *sems):
        isem = sems[:k]
        jsem = sems[k:]
        c = lax.axis_index("c")
        s = lax.axis_index("s")
        zero = jnp.zeros((16,), jnp.float32)

        def zfill(r, carry):
            for j in range(dc // 16):
                rows[0, r, pl.ds(j * 16, 16)] = zero
            return carry

        lax.fori_loop(0, _CHUNK, zfill, 0)
        row0, nchunks = _tile_chunks(s)

        def zcopy(q, carry):
            pltpu.sync_copy(rows.at[0],
                            acc.at[pl.ds(row0 + q * _CHUNK, _CHUNK)])
            return carry

        lax.fori_loop(0, nchunks, zcopy, 0)
        plsc.subcore_barrier()

        ebase = (c * (nb * _NSUB) + s * nb) * _EB

        def load_src(i, b):
            pltpu.async_copy(src_hbm.at[pl.ds(ebase + i * _EB, _EB)],
                             sidx.at[b, 0], isem[b])

        def load_dst(i, b):
            pltpu.async_copy(dst_hbm.at[pl.ds(ebase + i * _EB, _EB)],
                             didx.at[b, 0], jsem[b])

        def drain_src(b):
            pltpu.make_async_copy(src_hbm.at[pl.ds(0, _EB)], sidx.at[b, 0],
                                  isem[b]).wait()

        def drain_dst(b):
            pltpu.make_async_copy(dst_hbm.at[pl.ds(0, _EB)], didx.at[b, 0],
                                  jsem[b]).wait()

        def do_block(m, nbatch, prefetch_next):
            descs = []
            for b in range(nbatch):
                drain_src(b)
                descs.append(
                    pltpu.async_copy(h0.at[sidx.at[b, 0]], rows.at[b],
                                     isem[b]))
            for b in range(nbatch):
                descs[b].wait()
                if prefetch_next:
                    @pl.when(m * k + k + b < nb)
                    def _():
                        load_src(m * k + k + b, b)
            sdescs = []
            for b in range(nbatch):
                drain_dst(b)
                sdescs.append(
                    pltpu.async_copy(rows.at[b], acc.at[didx.at[b, 0]],
                                     jsem[b], add=True))
            for b in range(nbatch):
                sdescs[b].wait()
                if prefetch_next:
                    @pl.when(m * k + k + b < nb)
                    def _():
                        load_dst(m * k + k + b, b)

        for b in range(k):
            load_src(b, b)
            load_dst(b, b)

        def step(m, carry):
            do_block(m, k, True)
            return carry

        lax.fori_loop(0, nfull, step, 0)
        if tail:
            do_block(nfull, tail, False)
        plsc.subcore_barrier()

        @pl.when(c == 0)
        def _():
            def wb(q, carry):
                sl = pl.ds(row0 + q * _CHUNK, _CHUNK)
                pltpu.sync_copy(acc.at[sl], rows.at[0])
                pltpu.sync_copy(rows.at[0], out0.at[sl])
                return carry
            lax.fori_loop(0, nchunks, wb, 0)

        @pl.when(c == 1)
        def _():
            def wb(q, carry):
                sl = pl.ds(row0 + q * _CHUNK, _CHUNK)
                pltpu.sync_copy(acc.at[sl], rows.at[0])
                pltpu.sync_copy(rows.at[0], out1.at[sl])
                return carry
            lax.fori_loop(0, nchunks, wb, 0)

    return pl.kernel(
        body,
        out_type=(jax.ShapeDtypeStruct((_N, dc), jnp.float32),
                  jax.ShapeDtypeStruct((_N, dc), jnp.float32)),
        mesh=plsc.VectorSubcoreMesh(core_axis_name="c", subcore_axis_name="s"),
        scratch_types=(
            pltpu.VMEM((k, 1, _EB), jnp.int32),
            pltpu.VMEM((k, 1, _EB), jnp.int32),
            pltpu.VMEM((k, _EB, dc), jnp.float32),
            pltpu.VMEM_SHARED((_N, dc), jnp.float32),
        ) + (pltpu.SemaphoreType.DMA,) * (2 * k),
    )


_seg_sum = _make_seg_sum()


_BN = 1000


def _scale_body(deg0, deg1, x, dinv, xs):
    deg = deg0[...] + deg1[...] + 1.0
    di = lax.rsqrt(deg)
    dinv[...] = di
    xs[...] = di * x[...]


def _scale(deg0, deg1, x):
    return pl.pallas_call(
        _scale_body,
        grid=(_N // _BN,),
        in_specs=[
            pl.BlockSpec((_BN, 1), lambda i: (i, 0)),
            pl.BlockSpec((_BN, 1), lambda i: (i, 0)),
            pl.BlockSpec((_BN, 128), lambda i: (i, 0)),
        ],
        out_specs=[
            pl.BlockSpec((_BN, 1), lambda i: (i, 0)),
            pl.BlockSpec((_BN, 128), lambda i: (i, 0)),
        ],
        out_shape=[
            jax.ShapeDtypeStruct((_N, 1), jnp.float32),
            jax.ShapeDtypeStruct((_N, 128), jnp.float32),
        ],
    )(deg0, deg1, x)


def _zmm_body(*refs, pre_matmul, nagg):
    aggs = refs[:nagg]
    selfh, dinv, w, b, z, colsum, sumsq = refs[nagg:]
    i = pl.program_id(0)
    di = dinv[...]
    if nagg == 2:
        agg = aggs[0][...] + aggs[1][...]
    else:
        agg = jnp.concatenate([aggs[0][...] + aggs[1][...],
                               aggs[2][...] + aggs[3][...]], axis=1)
    pre = di * agg + di * selfh[...]
    if pre_matmul:
        zi = jnp.dot(pre, w[...], preferred_element_type=jnp.float32) + b[...]
    else:
        zi = pre + b[...]
    z[...] = zi

    @pl.when(i == 0)
    def _():
        colsum[...] = jnp.zeros_like(colsum)
        sumsq[...] = jnp.zeros_like(sumsq)

    colsum[...] += jnp.sum(zi, axis=0, keepdims=True)
    sumsq[...] += jnp.sum(zi * zi, keepdims=True).reshape(1, 1)


def _zmm(aggs, selfh, dinv, w, b, d_out, pre_matmul):
    dc = aggs[0].shape[1]
    din = selfh.shape[1]
    body = functools.partial(_zmm_body, pre_matmul=pre_matmul,
                             nagg=len(aggs))
    return pl.pallas_call(
        body,
        grid=(_N // _BN,),
        in_specs=[pl.BlockSpec((_BN, dc), lambda i: (i, 0))
                  for _ in aggs] + [
            pl.BlockSpec((_BN, din), lambda i: (i, 0)),
            pl.BlockSpec((_BN, 1), lambda i: (i, 0)),
            pl.BlockSpec(w.shape, lambda i: (0, 0)),
            pl.BlockSpec((1, d_out), lambda i: (0, 0)),
        ],
        out_specs=[
            pl.BlockSpec((_BN, d_out), lambda i: (i, 0)),
            pl.BlockSpec((1, d_out), lambda i: (0, 0)),
            pl.BlockSpec((1, 1), lambda i: (0, 0)),
        ],
        out_shape=[
            jax.ShapeDtypeStruct((_N, d_out), jnp.float32),
            jax.ShapeDtypeStruct((1, d_out), jnp.float32),
            jax.ShapeDtypeStruct((1, 1), jnp.float32),
        ],
    )(*aggs, selfh, dinv, w, b)


def _norm_mm_body(z, colsum, sumsq, dinv, w, *outs, split):
    mu = colsum[...] / _N
    var = sumsq[0, 0] / _N - jnp.sum(mu * mu)
    s = lax.rsqrt(1e-6 + var)
    h = jnp.maximum((z[...] - mu) * s, 0.0)
    t = jnp.dot(h, w[...], preferred_element_type=jnp.float32)
    hs = dinv[...] * t
    if split:
        half = t.shape[1] // 2
        outs[0][...] = hs[:, :half]
        outs[1][...] = hs[:, half:]
    else:
        outs[0][...] = hs


def _norm_mm(z, colsum, sumsq, dinv, w, split):
    d_in = z.shape[1]
    d_out = w.shape[1]
    if split:
        out_specs = [pl.BlockSpec((_BN, d_out // 2), lambda i: (i, 0)),
                     pl.BlockSpec((_BN, d_out // 2), lambda i: (i, 0))]
        out_shape = [jax.ShapeDtypeStruct((_N, d_out // 2), jnp.float32),
                     jax.ShapeDtypeStruct((_N, d_out // 2), jnp.float32)]
    else:
        out_specs = [pl.BlockSpec((_BN, d_out), lambda i: (i, 0))]
        out_shape = [jax.ShapeDtypeStruct((_N, d_out), jnp.float32)]
    return pl.pallas_call(
        functools.partial(_norm_mm_body, split=split),
        grid=(_N // _BN,),
        in_specs=[
            pl.BlockSpec((_BN, d_in), lambda i: (i, 0)),
            pl.BlockSpec((1, d_in), lambda i: (0, 0)),
            pl.BlockSpec((1, 1), lambda i: (0, 0)),
            pl.BlockSpec((_BN, 1), lambda i: (i, 0)),
            pl.BlockSpec(w.shape, lambda i: (0, 0)),
        ],
        out_specs=out_specs,
        out_shape=out_shape,
    )(z, colsum, sumsq, dinv, w)


def _final_body(p0, p1, hs, dinv, b, out):
    di = dinv[...]
    out[...] = di * (p0[...] + p1[...]) + di * hs[...] + b[...]


def _final(p0, p1, hs, dinv, b):
    dc = p0.shape[1]
    return pl.pallas_call(
        _final_body,
        grid=(_N // _BN,),
        in_specs=[
            pl.BlockSpec((_BN, dc), lambda i: (i, 0)),
            pl.BlockSpec((_BN, dc), lambda i: (i, 0)),
            pl.BlockSpec((_BN, dc), lambda i: (i, 0)),
            pl.BlockSpec((_BN, 1), lambda i: (i, 0)),
            pl.BlockSpec((1, dc), lambda i: (0, 0)),
        ],
        out_specs=pl.BlockSpec((_BN, dc), lambda i: (i, 0)),
        out_shape=jax.ShapeDtypeStruct((_N, dc), jnp.float32),
    )(p0, p1, hs, dinv, b)


def kernel(x, edge_index, W1, b1, W2, b2, W3, b3):
    src = edge_index[0]
    dst = edge_index[1]
    src2 = src.reshape(_E // _EB, 1, _EB)
    dst2 = dst.reshape(_E // _EB, 1, _EB)

    deg0, deg1 = _deg_kernel(dst)
    dinv, xs = _scale(deg0.reshape(_N, 1), deg1.reshape(_N, 1), x)

    p0, p1 = _seg_sum(xs, src, dst)
    z1, cs1, ss1 = _zmm((p0, p1), xs, dinv, W1, b1.reshape(1, -1),
                        256, pre_matmul=True)
    hs2_0, hs2_1 = _norm_mm(z1, cs1, ss1, dinv, W2, split=True)

    a0, a1 = _seg_sum(hs2_0, src, dst)
    hs2_1d, a0, a1 = lax.optimization_barrier((hs2_1, a0, a1))
    a2, a3 = _seg_sum(hs2_1d, src, dst)
    selfh2 = jnp.concatenate([hs2_0, hs2_1], axis=1)
    z2, cs2, ss2 = _zmm((a0, a1, a2, a3), selfh2, dinv, W2,
                        b2.reshape(1, -1), 256, pre_matmul=False)
    W3p = jnp.pad(W3, ((0, 0), (0, 88)))
    [hs3] = _norm_mm(z2, cs2, ss2, dinv, W3p, split=False)

    q0, q1 = _seg_sum(hs3, src, dst)
    b3p = jnp.pad(b3, (0, 88)).reshape(1, -1)
    outp = _final(q0, q1, hs3, dinv, b3p)
    return outp[:, :40]


# speedup vs baseline: 19.4801x; 1.0639x over previous
import functools

import jax
import jax.numpy as jnp
from jax import lax
from jax.experimental import pallas as pl
from jax.experimental.pallas import tpu as pltpu
from jax.experimental.pallas import tpu_sc as plsc

_N = 10000
_E = 320000
_EB = 80
_NSUB = 16
_NCORE = 2


def _zero_fill_vmem(ref, nwords):
    zero = jnp.zeros((16,), jnp.float32)

    def body(i, carry):
        ref[pl.ds(i * 16, 16)] = zero
        return carry

    lax.fori_loop(0, nwords // 16, body, 0)


_CHUNK = 80


def _tile_chunks(s):
    base = s * 640
    nchunks = jnp.where(s < 15, 8, 5)
    return base, nchunks


def _deg_kernel_body(dst_hbm, deg0_hbm, deg1_hbm, didx, ones, zbuf, acc,
                     dsem):
    c = lax.axis_index("c")
    s = lax.axis_index("s")
    one = jnp.full((16,), 1.0, jnp.float32)
    for j in range(_EB // 16):
        ones[pl.ds(j * 16, 16)] = one
    _zero_fill_vmem(zbuf, _CHUNK)
    row0, nchunks = _tile_chunks(s)

    def zcopy(k, carry):
        pltpu.sync_copy(zbuf, acc.at[pl.ds(row0 + k * _CHUNK, _CHUNK)])
        return carry

    lax.fori_loop(0, nchunks, zcopy, 0)
    plsc.subcore_barrier()
    eps = _E // (_NCORE * _NSUB)
    base = c * (_E // _NCORE) + s * eps

    nb = eps // _EB
    kd = 4

    def load(i, b):
        pltpu.async_copy(dst_hbm.at[pl.ds(base + i * _EB, _EB)],
                         didx.at[b, 0], dsem[b])

    def drain(b):
        pltpu.make_async_copy(dst_hbm.at[pl.ds(0, _EB)], didx.at[b, 0],
                              dsem[b]).wait()

    for b in range(kd):
        load(b, b)

    def body(m, carry):
        for b in range(kd):
            i = m * kd + b
            drain(b)
            pltpu.sync_copy(ones, acc.at[didx.at[b, 0]], add=True)

            @pl.when(i + kd < nb)
            def _():
                load(i + kd, b)
        return carry

    lax.fori_loop(0, nb // kd, body, 0)
    rem = nb - (nb // kd) * kd
    for b in range(rem):
        drain(b)
        pltpu.sync_copy(ones, acc.at[didx.at[b, 0]], add=True)
    plsc.subcore_barrier()

    @pl.when(c == 0)
    def _():
        def wb(k, carry):
            sl = pl.ds(row0 + k * _CHUNK, _CHUNK)
            pltpu.sync_copy(acc.at[sl], zbuf)
            pltpu.sync_copy(zbuf, deg0_hbm.at[sl])
            return carry
        lax.fori_loop(0, nchunks, wb, 0)

    @pl.when(c == 1)
    def _():
        def wb(k, carry):
            sl = pl.ds(row0 + k * _CHUNK, _CHUNK)
            pltpu.sync_copy(acc.at[sl], zbuf)
            pltpu.sync_copy(zbuf, deg1_hbm.at[sl])
            return carry
        lax.fori_loop(0, nchunks, wb, 0)


@functools.partial(
    pl.kernel,
    out_type=(jax.ShapeDtypeStruct((_N,), jnp.float32),
              jax.ShapeDtypeStruct((_N,), jnp.float32)),
    mesh=plsc.VectorSubcoreMesh(core_axis_name="c", subcore_axis_name="s"),
    scratch_types=(
        pltpu.VMEM((4, 1, _EB), jnp.int32),
        pltpu.VMEM((_EB,), jnp.float32),
        pltpu.VMEM((_CHUNK,), jnp.float32),
        pltpu.VMEM_SHARED((_N,), jnp.float32),
    ) + (pltpu.SemaphoreType.DMA,) * 4,
)
def _deg_kernel(dst_hbm, deg0_hbm, deg1_hbm, didx, ones, zbuf, acc, *dsems):
    _deg_kernel_body(dst_hbm, deg0_hbm, deg1_hbm, didx, ones, zbuf, acc,
                     dsems)


_DC = 128


def _make_seg_sum():
    dc = _DC
    k = 4
    nb = _E // (_NCORE * _NSUB) // _EB
    nfull = nb // k
    tail = nb - nfull * k

    def body(h0, src_hbm, dst_hbm, out0, out1, sidx, didx, rows, acc,
             *sems):
        isem = sems[:k]
        jsem = sems[k:]
        c = lax.axis_index("c")
        s = lax.axis_index("s")
        zero = jnp.zeros((16,), jnp.float32)

        def zfill(r, carry):
            for j in range(dc // 16):
                rows[0, r, pl.ds(j * 16, 16)] = zero
            return carry

        lax.fori_loop(0, _CHUNK, zfill, 0)
        row0, nchunks = _tile_chunks(s)

        def zcopy(q, carry):
            pltpu.sync_copy(rows.at[0],
                            acc.at[pl.ds(row0 + q * _CHUNK, _CHUNK)])
            return carry

        lax.fori_loop(0, nchunks, zcopy, 0)
        plsc.subcore_barrier()

        ebase = (c * (nb * _NSUB) + s * nb) * _EB

        def load_src(i, b):
            pltpu.async_copy(src_hbm.at[pl.ds(ebase + i * _EB, _EB)],
                             sidx.at[b, 0], isem[b])

        def load_dst(i, b):
            pltpu.async_copy(dst_hbm.at[pl.ds(ebase + i * _EB, _EB)],
                             didx.at[b, 0], jsem[b])

        def drain_src(b):
            pltpu.make_async_copy(src_hbm.at[pl.ds(0, _EB)], sidx.at[b, 0],
                                  isem[b]).wait()

        def drain_dst(b):
            pltpu.make_async_copy(dst_hbm.at[pl.ds(0, _EB)], didx.at[b, 0],
                                  jsem[b]).wait()

        def do_block(m, nbatch, prefetch_next):
            descs = []
            for b in range(nbatch):
                drain_src(b)
                descs.append(
                    pltpu.async_copy(h0.at[sidx.at[b, 0]], rows.at[b],
                                     isem[b]))
            for b in range(nbatch):
                descs[b].wait()
                if prefetch_next:
                    @pl.when(m * k + k + b < nb)
                    def _():
                        load_src(m * k + k + b, b)
            sdescs = []
            for b in range(nbatch):
                drain_dst(b)
                sdescs.append(
                    pltpu.async_copy(rows.at[b], acc.at[didx.at[b, 0]],
                                     jsem[b], add=True))
            for b in range(nbatch):
                sdescs[b].wait()
                if prefetch_next:
                    @pl.when(m * k + k + b < nb)
                    def _():
                        load_dst(m * k + k + b, b)

        for b in range(k):
            load_src(b, b)
            load_dst(b, b)

        def step(m, carry):
            do_block(m, k, True)
            return carry

        lax.fori_loop(0, nfull, step, 0)
        if tail:
            do_block(nfull, tail, False)
        plsc.subcore_barrier()

        @pl.when(c == 0)
        def _():
            def wb(q, carry):
                sl = pl.ds(row0 + q * _CHUNK, _CHUNK)
                pltpu.sync_copy(acc.at[sl], rows.at[0])
                pltpu.sync_copy(rows.at[0], out0.at[sl])
                return carry
            lax.fori_loop(0, nchunks, wb, 0)

        @pl.when(c == 1)
        def _():
            def wb(q, carry):
                sl = pl.ds(row0 + q * _CHUNK, _CHUNK)
                pltpu.sync_copy(acc.at[sl], rows.at[0])
                pltpu.sync_copy(rows.at[0], out1.at[sl])
                return carry
            lax.fori_loop(0, nchunks, wb, 0)

    return pl.kernel(
        body,
        out_type=(jax.ShapeDtypeStruct((_N, dc), jnp.float32),
                  jax.ShapeDtypeStruct((_N, dc), jnp.float32)),
        mesh=plsc.VectorSubcoreMesh(core_axis_name="c", subcore_axis_name="s"),
        scratch_types=(
            pltpu.VMEM((k, 1, _EB), jnp.int32),
            pltpu.VMEM((k, 1, _EB), jnp.int32),
            pltpu.VMEM((k, _EB, dc), jnp.float32),
            pltpu.VMEM_SHARED((_N, dc), jnp.float32),
        ) + (pltpu.SemaphoreType.DMA,) * (2 * k),
    )


_seg_sum = _make_seg_sum()


_BN = 1000


def _scale_body(deg0, deg1, x, dinv, xs):
    deg = deg0[...] + deg1[...] + 1.0
    di = lax.rsqrt(deg)
    dinv[...] = di
    xs[...] = di * x[...]


def _scale(deg0, deg1, x):
    return pl.pallas_call(
        _scale_body,
        grid=(_N // _BN,),
        in_specs=[
            pl.BlockSpec((_BN, 1), lambda i: (i, 0)),
            pl.BlockSpec((_BN, 1), lambda i: (i, 0)),
            pl.BlockSpec((_BN, 128), lambda i: (i, 0)),
        ],
        out_specs=[
            pl.BlockSpec((_BN, 1), lambda i: (i, 0)),
            pl.BlockSpec((_BN, 128), lambda i: (i, 0)),
        ],
        out_shape=[
            jax.ShapeDtypeStruct((_N, 1), jnp.float32),
            jax.ShapeDtypeStruct((_N, 128), jnp.float32),
        ],
    )(deg0, deg1, x)


def _zmm_body(*refs, pre_matmul, nagg):
    aggs = refs[:nagg]
    selfh, dinv, w, b, z, colsum, sumsq = refs[nagg:]
    i = pl.program_id(0)
    di = dinv[...]
    if nagg == 2:
        agg = aggs[0][...] + aggs[1][...]
    else:
        agg = jnp.concatenate([aggs[0][...] + aggs[1][...],
                               aggs[2][...] + aggs[3][...]], axis=1)
    pre = di * agg + di * selfh[...]
    if pre_matmul:
        zi = jnp.dot(pre, w[...], preferred_element_type=jnp.float32) + b[...]
    else:
        zi = pre + b[...]
    z[...] = zi

    @pl.when(i == 0)
    def _():
        colsum[...] = jnp.zeros_like(colsum)
        sumsq[...] = jnp.zeros_like(sumsq)

    colsum[...] += jnp.sum(zi, axis=0, keepdims=True)
    sumsq[...] += jnp.sum(zi * zi, keepdims=True).reshape(1, 1)


def _zmm(aggs, selfh, dinv, w, b, d_out, pre_matmul):
    dc = aggs[0].shape[1]
    din = selfh.shape[1]
    body = functools.partial(_zmm_body, pre_matmul=pre_matmul,
                             nagg=len(aggs))
    return pl.pallas_call(
        body,
        grid=(_N // _BN,),
        in_specs=[pl.BlockSpec((_BN, dc), lambda i: (i, 0))
                  for _ in aggs] + [
            pl.BlockSpec((_BN, din), lambda i: (i, 0)),
            pl.BlockSpec((_BN, 1), lambda i: (i, 0)),
            pl.BlockSpec(w.shape, lambda i: (0, 0)),
            pl.BlockSpec((1, d_out), lambda i: (0, 0)),
        ],
        out_specs=[
            pl.BlockSpec((_BN, d_out), lambda i: (i, 0)),
            pl.BlockSpec((1, d_out), lambda i: (0, 0)),
            pl.BlockSpec((1, 1), lambda i: (0, 0)),
        ],
        out_shape=[
            jax.ShapeDtypeStruct((_N, d_out), jnp.float32),
            jax.ShapeDtypeStruct((1, d_out), jnp.float32),
            jax.ShapeDtypeStruct((1, 1), jnp.float32),
        ],
    )(*aggs, selfh, dinv, w, b)


def _norm_mm_body(z, colsum, sumsq, dinv, w, *outs, split):
    mu = colsum[...] / _N
    var = sumsq[0, 0] / _N - jnp.sum(mu * mu)
    s = lax.rsqrt(1e-6 + var)
    h = jnp.maximum((z[...] - mu) * s, 0.0)
    t = jnp.dot(h, w[...], preferred_element_type=jnp.float32)
    hs = dinv[...] * t
    if split:
        half = t.shape[1] // 2
        outs[0][...] = hs[:, :half]
        outs[1][...] = hs[:, half:]
    else:
        outs[0][...] = hs


def _norm_mm(z, colsum, sumsq, dinv, w, split):
    d_in = z.shape[1]
    d_out = w.shape[1]
    if split:
        out_specs = [pl.BlockSpec((_BN, d_out // 2), lambda i: (i, 0)),
                     pl.BlockSpec((_BN, d_out // 2), lambda i: (i, 0))]
        out_shape = [jax.ShapeDtypeStruct((_N, d_out // 2), jnp.float32),
                     jax.ShapeDtypeStruct((_N, d_out // 2), jnp.float32)]
    else:
        out_specs = [pl.BlockSpec((_BN, d_out), lambda i: (i, 0))]
        out_shape = [jax.ShapeDtypeStruct((_N, d_out), jnp.float32)]
    return pl.pallas_call(
        functools.partial(_norm_mm_body, split=split),
        grid=(_N // _BN,),
        in_specs=[
            pl.BlockSpec((_BN, d_in), lambda i: (i, 0)),
            pl.BlockSpec((1, d_in), lambda i: (0, 0)),
            pl.BlockSpec((1, 1), lambda i: (0, 0)),
            pl.BlockSpec((_BN, 1), lambda i: (i, 0)),
            pl.BlockSpec(w.shape, lambda i: (0, 0)),
        ],
        out_specs=out_specs,
        out_shape=out_shape,
    )(z, colsum, sumsq, dinv, w)


def _final_body(p0, p1, hs, dinv, b, out):
    di = dinv[...]
    out[...] = di * (p0[...] + p1[...]) + di * hs[...] + b[...]


def _final(p0, p1, hs, dinv, b):
    dc = p0.shape[1]
    return pl.pallas_call(
        _final_body,
        grid=(_N // _BN,),
        in_specs=[
            pl.BlockSpec((_BN, dc), lambda i: (i, 0)),
            pl.BlockSpec((_BN, dc), lambda i: (i, 0)),
            pl.BlockSpec((_BN, dc), lambda i: (i, 0)),
            pl.BlockSpec((_BN, 1), lambda i: (i, 0)),
            pl.BlockSpec((1, dc), lambda i: (0, 0)),
        ],
        out_specs=pl.BlockSpec((_BN, dc), lambda i: (i, 0)),
        out_shape=jax.ShapeDtypeStruct((_N, dc), jnp.float32),
    )(p0, p1, hs, dinv, b)


def kernel(x, edge_index, W1, b1, W2, b2, W3, b3):
    src = edge_index[0]
    dst = edge_index[1]
    src2 = src.reshape(_E // _EB, 1, _EB)
    dst2 = dst.reshape(_E // _EB, 1, _EB)

    deg0, deg1 = _deg_kernel(dst)
    dinv, xs = _scale(deg0.reshape(_N, 1), deg1.reshape(_N, 1), x)

    p0, p1 = _seg_sum(xs, src, dst)
    z1, cs1, ss1 = _zmm((p0, p1), xs, dinv, W1, b1.reshape(1, -1),
                        256, pre_matmul=True)
    hs2_0, hs2_1 = _norm_mm(z1, cs1, ss1, dinv, W2, split=True)

    a0, a1 = _seg_sum(hs2_0, src, dst)
    hs2_1d, a0, a1 = lax.optimization_barrier((hs2_1, a0, a1))
    a2, a3 = _seg_sum(hs2_1d, src, dst)
    selfh2 = jnp.concatenate([hs2_0, hs2_1], axis=1)
    z2, cs2, ss2 = _zmm((a0, a1, a2, a3), selfh2, dinv, W2,
                        b2.reshape(1, -1), 256, pre_matmul=False)
    W3p = jnp.pad(W3, ((0, 0), (0, 88)))
    [hs3] = _norm_mm(z2, cs2, ss2, dinv, W3p, split=False)

    q0, q1 = _seg_sum(hs3, src, dst)
    b3p = jnp.pad(b3, (0, 88)).reshape(1, -1)
    outp = _final(q0, q1, hs3, dinv, b3p)
    return outp[:, :40]
